# baseline (speedup 1.0000x reference)
import functools

import jax
import jax.numpy as jnp
from jax import lax
from jax.experimental import pallas as pl
from jax.experimental.pallas import tpu as pltpu
from jax.experimental.pallas import tpu_sc as plsc

B = 16384
F = 26
V = 100001
D = 32

NC = 2
NS = 16
NW = NC * NS
VEC_PW = F * D // NW

CA = 50048
CB = V - CA

QCH = 4096
NQ = B // QCH


def _emb_body(tab_hbm, idx_hbm, out_hbm, bufx, bufy, idxv, outv,
              xsem, ysem, osem):
  wid = lax.axis_index("s") * NC + lax.axis_index("c")

  vid0 = wid * VEC_PW
  f0 = vid0 // D
  d0 = lax.rem(vid0, D)
  pltpu.async_copy(tab_hbm.at[f0, d0, pl.ds(0, CA)], bufx, xsem)
  pltpu.async_copy(tab_hbm.at[f0, d0, pl.ds(CA, CB)], bufy, ysem)

  def vec_step(k, _):
    vid = wid * VEC_PW + k
    f = vid // D
    d = lax.rem(vid, D)

    @pl.when(jnp.logical_or(k == 0, f != (vid - 1) // D))
    def _():
      pltpu.sync_copy(idx_hbm.at[f], idxv)

    pltpu.make_async_copy(tab_hbm.at[f, d, pl.ds(0, CA)], bufx, xsem).wait()

    for q in range(NQ):
      ob = q % 2

      if q >= 2:
        pltpu.make_async_copy(
            outv.at[ob], out_hbm.at[f, d, pl.ds(0, QCH)], osem).wait()
      else:
        @pl.when(k > 0)
        def _():
          pltpu.make_async_copy(
              outv.at[ob], out_hbm.at[f, d, pl.ds(0, QCH)], osem).wait()

      def stepA(i, _, q=q, ob=ob):
        base = i * 256
        for j in range(16):
          idx16 = idxv[pl.ds(q * QCH + base + j * 16, 16)]
          ia = jnp.minimum(idx16, CA - 1)
          outv[ob, pl.ds(base + j * 16, 16)] = plsc.load_gather(bufx, [ia])
        return 0

      lax.fori_loop(0, QCH // 256, stepA, 0)

      if q == NQ - 1:
        @pl.when(k + 1 < VEC_PW)
        def _():
          nvid = vid + 1
          pltpu.async_copy(
              tab_hbm.at[nvid // D, lax.rem(nvid, D), pl.ds(0, CA)],
              bufx, xsem)
      if q == 0:
        pltpu.make_async_copy(
            tab_hbm.at[f, d, pl.ds(CA, CB)], bufy, ysem).wait()

      def stepB(i, _, q=q, ob=ob):
        base = i * 256
        for j in range(16):
          idx16 = idxv[pl.ds(q * QCH + base + j * 16, 16)]
          ib = jnp.minimum(jnp.maximum(idx16 - CA, 0), CB - 1)
          gb = plsc.load_gather(bufy, [ib])
          prev = outv[ob, pl.ds(base + j * 16, 16)]
          outv[ob, pl.ds(base + j * 16, 16)] = jnp.where(idx16 >= CA, gb, prev)
        return 0

      lax.fori_loop(0, QCH // 256, stepB, 0)

      if q == NQ - 1:
        @pl.when(k + 1 < VEC_PW)
        def _():
          nvid = vid + 1
          pltpu.async_copy(
              tab_hbm.at[nvid // D, lax.rem(nvid, D), pl.ds(CA, CB)],
              bufy, ysem)

      pltpu.async_copy(
          outv.at[ob], out_hbm.at[f, d, pl.ds(q * QCH, QCH)], osem)
    return 0

  lax.fori_loop(0, VEC_PW, vec_step, 0)

  for _ in range(2):
    pltpu.make_async_copy(
        outv.at[0], out_hbm.at[0, 0, pl.ds(0, QCH)], osem).wait()


@jax.jit
def kernel(indices, tables):
  tabT = jnp.transpose(tables, (0, 2, 1))
  idxT = jnp.transpose(indices, (1, 0))
  mesh = plsc.VectorSubcoreMesh(
      core_axis_name="c", subcore_axis_name="s", num_cores=NC, num_subcores=NS)
  run = functools.partial(
      pl.kernel,
      out_type=jax.ShapeDtypeStruct((F, D, B), jnp.float32),
      mesh=mesh,
      scratch_types=[
          pltpu.VMEM((CA,), jnp.float32),
          pltpu.VMEM((CB,), jnp.float32),
          pltpu.VMEM((B,), jnp.int32),
          pltpu.VMEM((2, QCH), jnp.float32),
          pltpu.SemaphoreType.DMA,
          pltpu.SemaphoreType.DMA,
          pltpu.SemaphoreType.DMA,
      ],
      compiler_params=pltpu.CompilerParams(needs_layout_passes=False),
  )(_emb_body)
  outT = run(tabT, idxT)
  return jnp.transpose(outT, (2, 0, 1))


# speedup vs baseline: 1.6855x; 1.6855x over previous
import functools

import jax
import jax.numpy as jnp
from jax import lax
from jax.experimental import pallas as pl
from jax.experimental.pallas import tpu as pltpu
from jax.experimental.pallas import tpu_sc as plsc

B = 16384
F = 26
V = 100001
D = 32

NC = 2
NS = 16
NW = NC * NS
VEC_PW = F * D // NW

QCH = 4096
NQ = B // QCH


def _emb_body(tab_hbm, idx_hbm, out_hbm, vvec, idxv, outv, vsem, osem):
  wid = lax.axis_index("s") * NC + lax.axis_index("c")

  f_prev = None
  nwrites = 0
  for k in range(VEC_PW):
    vid = wid * VEC_PW + k
    f = vid // D
    d = lax.rem(vid, D)

    pltpu.async_copy(tab_hbm.at[f, d], vvec, vsem)
    if f_prev is None:
      pltpu.sync_copy(idx_hbm.at[f], idxv)
    else:
      @pl.when(f != f_prev)
      def _():
        pltpu.sync_copy(idx_hbm.at[f], idxv)
    f_prev = f
    pltpu.make_async_copy(tab_hbm.at[f, d], vvec, vsem).wait()

    for q in range(NQ):
      buf = nwrites % 2
      if nwrites >= 2:
        pltpu.make_async_copy(
            outv.at[buf], out_hbm.at[f, d, pl.ds(0, QCH)], osem).wait()

      def qstep(i, _, q=q, buf=buf):
        base = i * 256
        for j in range(16):
          idx16 = idxv[pl.ds(q * QCH + base + j * 16, 16)]
          outv[buf, pl.ds(base + j * 16, 16)] = plsc.load_gather(
              vvec, [idx16])
        return 0

      lax.fori_loop(0, QCH // 256, qstep, 0)
      pltpu.async_copy(
          outv.at[buf], out_hbm.at[f, d, pl.ds(q * QCH, QCH)], osem)
      nwrites += 1

  for _ in range(2):
    pltpu.make_async_copy(
        outv.at[0], out_hbm.at[0, 0, pl.ds(0, QCH)], osem).wait()


@jax.jit
def kernel(indices, tables):
  tabT = jnp.transpose(tables, (0, 2, 1))
  idxT = jnp.transpose(indices, (1, 0))
  mesh = plsc.VectorSubcoreMesh(
      core_axis_name="c", subcore_axis_name="s", num_cores=NC, num_subcores=NS)
  run = functools.partial(
      pl.kernel,
      out_type=jax.ShapeDtypeStruct((F, D, B), jnp.float32),
      mesh=mesh,
      scratch_types=[
          pltpu.VMEM((V,), jnp.float32),
          pltpu.VMEM((B,), jnp.int32),
          pltpu.VMEM((2, QCH), jnp.float32),
          pltpu.SemaphoreType.DMA,
          pltpu.SemaphoreType.DMA,
      ],
      compiler_params=pltpu.CompilerParams(needs_layout_passes=False),
  )(_emb_body)
  outT = run(tabT, idxT)
  return jnp.transpose(outT, (2, 0, 1))


# speedup vs baseline: 2.4481x; 1.4524x over previous
import functools

import jax
import jax.numpy as jnp
from jax import lax
from jax.experimental import pallas as pl
from jax.experimental.pallas import tpu as pltpu
from jax.experimental.pallas import tpu_sc as plsc

B = 16384
F = 26
V = 100001
D = 32

NC = 2
NS = 16
NW = NC * NS
VEC_PW = F * D // NW

HALF = B // 2


def _emb_body(tab_hbm, idx_hbm, out_hbm, vvec, idxv, outv, vsem):
  wid = lax.axis_index("s") * NC + lax.axis_index("c")

  vid0 = wid * VEC_PW
  pltpu.async_copy(tab_hbm.at[vid0 // D, lax.rem(vid0, D)], vvec, vsem)

  for k in range(VEC_PW):
    vid = wid * VEC_PW + k
    f = vid // D
    d = lax.rem(vid, D)

    if k == 0:
      pltpu.sync_copy(idx_hbm.at[f], idxv)
    else:
      @pl.when(f != (vid - 1) // D)
      def _():
        pltpu.sync_copy(idx_hbm.at[f], idxv)

    pltpu.make_async_copy(tab_hbm.at[f, d], vvec, vsem).wait()

    for half in range(2):
      def step(i, _, half=half):
        base = i * 256
        for j in range(16):
          idx16 = idxv[pl.ds(half * HALF + base + j * 16, 16)]
          outv[pl.ds(base + j * 16, 16)] = plsc.load_gather(vvec, [idx16])
        return 0

      lax.fori_loop(0, HALF // 256, step, 0)

      if half == 1 and k + 1 < VEC_PW:
        nvid = vid + 1
        pltpu.async_copy(
            tab_hbm.at[nvid // D, lax.rem(nvid, D)], vvec, vsem)

      pltpu.sync_copy(outv, out_hbm.at[f, d, pl.ds(half * HALF, HALF)])


@jax.jit
def kernel(indices, tables):
  tabT = jnp.transpose(tables, (0, 2, 1))
  idxT = jnp.transpose(indices, (1, 0))
  mesh = plsc.VectorSubcoreMesh(
      core_axis_name="c", subcore_axis_name="s", num_cores=NC, num_subcores=NS)
  run = functools.partial(
      pl.kernel,
      out_type=jax.ShapeDtypeStruct((F, D, B), jnp.float32),
      mesh=mesh,
      scratch_types=[
          pltpu.VMEM((V,), jnp.float32),
          pltpu.VMEM((B,), jnp.int32),
          pltpu.VMEM((HALF,), jnp.float32),
          pltpu.SemaphoreType.DMA,
      ],
      compiler_params=pltpu.CompilerParams(needs_layout_passes=False),
  )(_emb_body)
  outT = run(tabT, idxT)
  return jnp.transpose(outT, (2, 0, 1))
